# Initial kernel scaffold; baseline (speedup 1.0000x reference)
import jax
import jax.numpy as jnp
from jax.experimental import pallas as pl


def kernel(x, xt, Wq, Wk, Wv, Wout, w1, b1, bn_w, bn_b, w2, b2):
    raise NotImplementedError("write your pallas kernel here")


# speedup vs baseline: 8.7025x; 8.7025x over previous
import jax
import jax.numpy as jnp
from jax import lax
from jax.experimental import pallas as pl
from jax.experimental.pallas import tpu as pltpu

_B, _IN, _N = 8, 64, 2048
_M = 512
_OUT = 128
_C = _IN + 3


def _fps_body(x_ref, far0_ref, cidx_ref):
    iota_n = lax.broadcasted_iota(jnp.int32, (_B, _N), 1)
    iota_m = lax.broadcasted_iota(jnp.int32, (_B, _M), 1)
    far0 = far0_ref[...]
    dist0 = jnp.full((_B, _N), 1e10, dtype=jnp.float32)
    cacc0 = jnp.zeros((_B, _M), dtype=jnp.int32)

    def step(i, st):
        dist, far, cacc = st
        cacc = jnp.where(iota_m == i, far, cacc)
        x = x_ref[...]
        onehot = (iota_n == far).astype(jnp.float32)
        sel = jnp.sum(x * onehot[:, None, :], axis=2, keepdims=True)
        d = jnp.sum((x - sel) ** 2, axis=1)
        dist = jnp.minimum(dist, d)
        mx = jnp.max(dist, axis=1, keepdims=True)
        cand = jnp.where(dist == mx, iota_n, _N)
        far = jnp.min(cand, axis=1, keepdims=True)
        return dist, far, cacc

    _, _, cacc = lax.fori_loop(0, _M, step, (dist0, far0, cacc0))
    cidx_ref[...] = cacc


def _fwd_body(x_ref, cidx_ref, wq_ref, wk_ref, wv_ref, wout_ref, w1_ref,
              b1_ref, bnw_ref, bnb_ref, w2_ref, b2_ref, out_ref, h_scr):
    f32 = jnp.float32
    iota_nm = lax.broadcasted_iota(jnp.int32, (_N, _M), 0)
    scale = 1.0 / (_OUT ** 0.5)
    hi = jax.lax.Precision.HIGHEST

    s1 = jnp.zeros((_OUT, 1), dtype=f32)
    for b in range(_B):
        xb = x_ref[b]
        row = cidx_ref[b:b + 1, :]
        onehot = (iota_nm == row).astype(f32)
        cent = jax.lax.dot(xb, onehot, precision=hi)
        q = jax.lax.dot(wq_ref[...], cent)
        k = jax.lax.dot(wk_ref[...], xb)
        v = jax.lax.dot(wv_ref[...], xb)
        logits = lax.dot_general(q, k, (((0,), (0,)), ((), ()))) * scale
        probs = jax.nn.softmax(logits, axis=-1)
        att = lax.dot_general(v, probs, (((1,), (1,)), ((), ())))
        att = jax.lax.dot(wout_ref[...], att)
        h = jax.lax.dot(w1_ref[...], att) + b1_ref[...]
        h_scr[b] = h
        s1 = s1 + jnp.sum(h, axis=1, keepdims=True)

    mean = s1 * (1.0 / (_B * _M))
    s2 = jnp.zeros((_OUT, 1), dtype=f32)
    for b in range(_B):
        dh = h_scr[b] - mean
        s2 = s2 + jnp.sum(dh * dh, axis=1, keepdims=True)
    var = s2 * (1.0 / (_B * _M))
    inv = 1.0 / jnp.sqrt(var + 1e-5)

    for b in range(_B):
        hn = (h_scr[b] - mean) * inv
        hn = hn * bnw_ref[...] + bnb_ref[...]
        hn = jnp.where(hn >= 0, hn, 0.2 * hn)
        out_ref[b] = jax.lax.dot(w2_ref[...], hn) + b2_ref[...]


def kernel(x, xt, Wq, Wk, Wv, Wout, w1, b1, bn_w, bn_b, w2, b2):
    xcat = jnp.concatenate([x, xt], axis=1)
    far0 = jax.random.randint(jax.random.key(42), (_B,), 0, _N).astype(
        jnp.int32).reshape(_B, 1)

    vmem = pl.BlockSpec(memory_space=pltpu.VMEM)
    cidx = pl.pallas_call(
        _fps_body,
        out_shape=jax.ShapeDtypeStruct((_B, _M), jnp.int32),
        in_specs=[vmem, vmem],
        out_specs=vmem,
    )(xcat, far0)

    out = pl.pallas_call(
        _fwd_body,
        out_shape=jax.ShapeDtypeStruct((_B, _OUT, _M), jnp.float32),
        in_specs=[vmem] * 12,
        out_specs=vmem,
        scratch_shapes=[pltpu.VMEM((_B, _OUT, _M), jnp.float32)],
    )(xcat, cidx, Wq, Wk, Wv, Wout, w1,
      b1.reshape(_OUT, 1), bn_w.reshape(_OUT, 1), bn_b.reshape(_OUT, 1),
      w2, b2.reshape(_OUT, 1))
    return (out, cidx)


# speedup vs baseline: 10.4515x; 1.2010x over previous
import jax
import jax.numpy as jnp
from jax import lax
from jax.experimental import pallas as pl
from jax.experimental.pallas import tpu as pltpu

_B, _IN, _N = 8, 64, 2048
_M = 512
_OUT = 128
_C = _IN + 3


def _fps_body(x_ref, far0_ref, cidx_ref):
    iota_n = lax.broadcasted_iota(jnp.int32, (_B, _N), 1)
    iota_m = lax.broadcasted_iota(jnp.int32, (_B, _M), 1)
    far0 = far0_ref[...]
    dist0 = jnp.full((_B, _N), 1e10, dtype=jnp.float32)
    cacc0 = jnp.zeros((_B, _M), dtype=jnp.int32)
    hi = jax.lax.Precision.HIGHEST

    def step(i, st):
        dist, far, cacc = st
        cacc = jnp.where(iota_m == i, far, cacc)
        onehot = (iota_n == far).astype(jnp.float32)
        rows = []
        for b in range(_B):
            xb = x_ref[b]
            selb = lax.dot_general(xb, onehot[b:b + 1, :],
                                   (((1,), (1,)), ((), ())),
                                   precision=hi)
            rows.append(jnp.sum((xb - selb) ** 2, axis=0, keepdims=True))
        d = jnp.concatenate(rows, axis=0)
        dist = jnp.minimum(dist, d)
        mx = jnp.max(dist, axis=1, keepdims=True)
        cand = jnp.where(dist == mx, iota_n, _N)
        far = jnp.min(cand, axis=1, keepdims=True)
        return dist, far, cacc

    _, _, cacc = lax.fori_loop(0, _M, step, (dist0, far0, cacc0))
    cidx_ref[...] = cacc


def _fwd_body(x_ref, cidx_ref, wq_ref, wk_ref, wv_ref, wout_ref, w1_ref,
              b1_ref, bnw_ref, bnb_ref, w2_ref, b2_ref, out_ref, h_scr):
    f32 = jnp.float32
    iota_nm = lax.broadcasted_iota(jnp.int32, (_N, _M), 0)
    scale = 1.0 / (_OUT ** 0.5)
    hi = jax.lax.Precision.HIGHEST

    s1 = jnp.zeros((_OUT, 1), dtype=f32)
    for b in range(_B):
        xb = x_ref[b]
        row = cidx_ref[b:b + 1, :]
        onehot = (iota_nm == row).astype(f32)
        cent = jax.lax.dot(xb, onehot, precision=hi)
        q = jax.lax.dot(wq_ref[...], cent)
        k = jax.lax.dot(wk_ref[...], xb)
        v = jax.lax.dot(wv_ref[...], xb)
        logits = lax.dot_general(q, k, (((0,), (0,)), ((), ()))) * scale
        probs = jax.nn.softmax(logits, axis=-1)
        att = lax.dot_general(v, probs, (((1,), (1,)), ((), ())))
        att = jax.lax.dot(wout_ref[...], att)
        h = jax.lax.dot(w1_ref[...], att) + b1_ref[...]
        h_scr[b] = h
        s1 = s1 + jnp.sum(h, axis=1, keepdims=True)

    mean = s1 * (1.0 / (_B * _M))
    s2 = jnp.zeros((_OUT, 1), dtype=f32)
    for b in range(_B):
        dh = h_scr[b] - mean
        s2 = s2 + jnp.sum(dh * dh, axis=1, keepdims=True)
    var = s2 * (1.0 / (_B * _M))
    inv = 1.0 / jnp.sqrt(var + 1e-5)

    for b in range(_B):
        hn = (h_scr[b] - mean) * inv
        hn = hn * bnw_ref[...] + bnb_ref[...]
        hn = jnp.where(hn >= 0, hn, 0.2 * hn)
        out_ref[b] = jax.lax.dot(w2_ref[...], hn) + b2_ref[...]


def kernel(x, xt, Wq, Wk, Wv, Wout, w1, b1, bn_w, bn_b, w2, b2):
    xcat = jnp.concatenate([x, xt], axis=1)
    far0 = jax.random.randint(jax.random.key(42), (_B,), 0, _N).astype(
        jnp.int32).reshape(_B, 1)

    vmem = pl.BlockSpec(memory_space=pltpu.VMEM)
    cidx = pl.pallas_call(
        _fps_body,
        out_shape=jax.ShapeDtypeStruct((_B, _M), jnp.int32),
        in_specs=[vmem, vmem],
        out_specs=vmem,
    )(xcat, far0)

    out = pl.pallas_call(
        _fwd_body,
        out_shape=jax.ShapeDtypeStruct((_B, _OUT, _M), jnp.float32),
        in_specs=[vmem] * 12,
        out_specs=vmem,
        scratch_shapes=[pltpu.VMEM((_B, _OUT, _M), jnp.float32)],
    )(xcat, cidx, Wq, Wk, Wv, Wout, w1,
      b1.reshape(_OUT, 1), bn_w.reshape(_OUT, 1), bn_b.reshape(_OUT, 1),
      w2, b2.reshape(_OUT, 1))
    return (out, cidx)


# speedup vs baseline: 11.8198x; 1.1309x over previous
import functools

import jax
import jax.numpy as jnp
from jax import lax
from jax.experimental import pallas as pl
from jax.experimental.pallas import tpu as pltpu
from jax.experimental.pallas import tpu_sc as plsc

_B, _IN, _N = 8, 64, 2048
_M = 512
_OUT = 128
_C = _IN + 3


_CP = 72


def _fps_body(x_ref, far0_ref, cidx_ref):
    iota_n = lax.broadcasted_iota(jnp.int32, (_B, _N), 1)
    iota_1 = lax.broadcasted_iota(jnp.int32, (1, _N), 1)
    iota_m = lax.broadcasted_iota(jnp.int32, (_B, _M), 1)
    far0 = far0_ref[...]
    dist0 = jnp.full((_B, _N), 1e10, dtype=jnp.float32)
    cacc0 = jnp.zeros((_B, _M), dtype=jnp.int32)

    def step(i, st):
        dist, far, cacc = st
        cacc = jnp.where(iota_m == i, far, cacc)
        rows = []
        for b in range(_B):
            ohb = (iota_1 == far[b:b + 1, :]).astype(jnp.float32)
            dacc = jnp.zeros((8, _N), dtype=jnp.float32)
            for t in range(_CP // 8):
                tile = x_ref[b, 8 * t:8 * t + 8, :]
                selt = jnp.sum(tile * ohb, axis=1, keepdims=True)
                diff = tile - selt
                dacc = dacc + diff * diff
            rows.append(jnp.sum(dacc, axis=0, keepdims=True))
        d = jnp.concatenate(rows, axis=0)
        dist = jnp.minimum(dist, d)
        mx = jnp.max(dist, axis=1, keepdims=True)
        cand = jnp.where(dist == mx, iota_n, _N)
        far = jnp.min(cand, axis=1, keepdims=True)
        return dist, far, cacc

    _, _, cacc = lax.fori_loop(0, _M, step, (dist0, far0, cacc0))
    cidx_ref[...] = cacc


_DP = 128


def _sc_gather(table, idx):
    info = plsc.get_sparse_core_info()
    nc, ns = info.num_cores, info.num_subcores
    nw = nc * ns
    rows_total = _B * _M
    per_w = rows_total // nw
    mesh = plsc.VectorSubcoreMesh(core_axis_name="c", subcore_axis_name="s")

    @functools.partial(
        pl.kernel, mesh=mesh,
        out_type=jax.ShapeDtypeStruct((rows_total, _DP), jnp.float32),
        scratch_types=[
            pltpu.VMEM((per_w,), jnp.int32),
            pltpu.VMEM((per_w, _DP), jnp.float32),
            pltpu.SemaphoreType.DMA,
        ],
    )
    def gather_kernel(table_hbm, idx_hbm, out_hbm, idx_v, rows_v, sem):
        wid = lax.axis_index("s") * nc + lax.axis_index("c")
        base = wid * per_w
        pltpu.sync_copy(idx_hbm.at[pl.ds(base, per_w)], idx_v)
        pltpu.async_copy(table_hbm.at[idx_v], rows_v, sem).wait()
        pltpu.sync_copy(rows_v, out_hbm.at[pl.ds(base, per_w)])

    return gather_kernel(table, idx)


def _fwd_body(x_ref, cent_ref, wq_ref, wk_ref, wv_ref, wout_ref, w1_ref,
              b1_ref, bnw_ref, bnb_ref, w2_ref, b2_ref, out_ref, h_scr):
    f32 = jnp.float32
    scale = 1.0 / (_OUT ** 0.5)

    s1 = jnp.zeros((_OUT, 1), dtype=f32)
    for b in range(_B):
        xb = x_ref[b]
        cent = cent_ref[b][:, 0:_C]
        q = lax.dot_general(wq_ref[...], cent,
                            (((1,), (1,)), ((), ())))
        k = jax.lax.dot(wk_ref[...], xb)
        v = jax.lax.dot(wv_ref[...], xb)
        logits = lax.dot_general(q, k, (((0,), (0,)), ((), ()))) * scale
        probs = jax.nn.softmax(logits, axis=-1)
        att = lax.dot_general(v, probs, (((1,), (1,)), ((), ())))
        att = jax.lax.dot(wout_ref[...], att)
        h = jax.lax.dot(w1_ref[...], att) + b1_ref[...]
        h_scr[b] = h
        s1 = s1 + jnp.sum(h, axis=1, keepdims=True)

    mean = s1 * (1.0 / (_B * _M))
    s2 = jnp.zeros((_OUT, 1), dtype=f32)
    for b in range(_B):
        dh = h_scr[b] - mean
        s2 = s2 + jnp.sum(dh * dh, axis=1, keepdims=True)
    var = s2 * (1.0 / (_B * _M))
    inv = 1.0 / jnp.sqrt(var + 1e-5)

    for b in range(_B):
        hn = (h_scr[b] - mean) * inv
        hn = hn * bnw_ref[...] + bnb_ref[...]
        hn = jnp.where(hn >= 0, hn, 0.2 * hn)
        out_ref[b] = jax.lax.dot(w2_ref[...], hn) + b2_ref[...]


def kernel(x, xt, Wq, Wk, Wv, Wout, w1, b1, bn_w, bn_b, w2, b2):
    xcat = jnp.concatenate([x, xt], axis=1)
    far0 = jax.random.randint(jax.random.key(42), (_B,), 0, _N).astype(
        jnp.int32).reshape(_B, 1)

    vmem = pl.BlockSpec(memory_space=pltpu.VMEM)
    xpad = jnp.concatenate(
        [xcat, jnp.zeros((_B, _CP - _C, _N), dtype=xcat.dtype)], axis=1)
    cidx = pl.pallas_call(
        _fps_body,
        out_shape=jax.ShapeDtypeStruct((_B, _M), jnp.int32),
        in_specs=[vmem, vmem],
        out_specs=vmem,
    )(xpad, far0)

    xrows = jnp.concatenate(
        [jnp.transpose(xcat, (0, 2, 1)),
         jnp.zeros((_B, _N, _DP - _C), dtype=xcat.dtype)],
        axis=2).reshape(_B * _N, _DP)
    flat_idx = (cidx + (jnp.arange(_B, dtype=jnp.int32) * _N)[:, None]
                ).reshape(_B * _M)
    cent_rows = _sc_gather(xrows, flat_idx).reshape(_B, _M, _DP)

    out = pl.pallas_call(
        _fwd_body,
        out_shape=jax.ShapeDtypeStruct((_B, _OUT, _M), jnp.float32),
        in_specs=[vmem] * 12,
        out_specs=vmem,
        scratch_shapes=[pltpu.VMEM((_B, _OUT, _M), jnp.float32)],
    )(xcat, cent_rows, Wq, Wk, Wv, Wout, w1,
      b1.reshape(_OUT, 1), bn_w.reshape(_OUT, 1), bn_b.reshape(_OUT, 1),
      w2, b2.reshape(_OUT, 1))
    return (out, cidx)
